# Initial kernel scaffold; baseline (speedup 1.0000x reference)
import functools

import jax
import jax.numpy as jnp
from jax import lax
from jax.experimental import pallas as pl
from jax.experimental.pallas import tpu as pltpu
from jax.experimental.pallas import tpu_sc as plsc

_N_ITEMS = 50000
_EPI = 16
_D = 64
_NW = 32
_C = 32
_NPT = 1568
_NP = _NW * _NPT
_NCH = _NPT // _C


def _body(ent_hbm, idx_hbm, att_hbm, out_hbm,
          idx_v, rows_v, att_v, base_v, w_v, out_v, sem):
    wid = lax.axis_index("s") * 2 + lax.axis_index("c")
    lane = lax.iota(jnp.int32, 16)

    def chunk(k, carry):
        base = wid * _NPT + k * _C
        pltpu.sync_copy(idx_hbm.at[pl.ds(base // 8, 4)], idx_v)
        gathers = [
            pltpu.async_copy(ent_hbm.at[idx_v.at[j]],
                             rows_v.at[pl.ds(j * 128, 128)], sem)
            for j in range(4)
        ]
        pltpu.sync_copy(att_hbm.at[pl.ds(base, _C)], att_v)
        pltpu.sync_copy(ent_hbm.at[pl.ds(base, _C)], base_v)

        def softmax_item(i, c):
            a = att_v[i]
            m = jnp.max(a)
            ev = jnp.exp(a - m)
            s = jnp.sum(ev)
            w_v[i] = ev / s
            return c

        lax.fori_loop(0, _C, softmax_item, 0)

        for g in gathers:
            g.wait()

        def item(i, c):
            w = w_v[i]
            acc = [base_v[i, pl.ds(d * 16, 16)] for d in range(4)]
            r0 = i * _EPI
            for e in range(_EPI):
                we = jnp.sum(jnp.where(lane == e, w, 0.0))
                for d in range(4):
                    acc[d] = acc[d] + we * rows_v[r0 + e, pl.ds(d * 16, 16)]
            for d in range(4):
                out_v[i, pl.ds(d * 16, 16)] = acc[d]
            return c

        lax.fori_loop(0, _C, item, 0)

        pltpu.sync_copy(out_v, out_hbm.at[pl.ds(base, _C)])
        return carry

    lax.fori_loop(0, _NCH, chunk, 0)


@functools.partial(
    pl.kernel,
    out_type=jax.ShapeDtypeStruct((_NP, _D), jnp.float32),
    mesh=plsc.VectorSubcoreMesh(core_axis_name="c", subcore_axis_name="s",
                                num_cores=2, num_subcores=16),
    scratch_types=[
        pltpu.VMEM((4, 128), jnp.int32),
        pltpu.VMEM((_C * _EPI, _D), jnp.float32),
        pltpu.VMEM((_C, _EPI), jnp.float32),
        pltpu.VMEM((_C, _D), jnp.float32),
        pltpu.VMEM((_C, _EPI), jnp.float32),
        pltpu.VMEM((_C, _D), jnp.float32),
        pltpu.SemaphoreType.DMA,
    ],
)
def _sc_gather_attend(ent, idx, att, out, *scratch):
    _body(ent, idx, att, out, *scratch)


def kernel(item_entities, entiEmbs, att, uEmbeds):
    ie = item_entities.astype(jnp.int32)
    pad = _NP - _N_ITEMS
    ie_p = jnp.pad(ie, ((0, pad), (0, 0)))
    att_p = jnp.pad(att, ((0, pad), (0, 0)))
    idx2d = ie_p.reshape(_NP * _EPI // 128, 128)
    out = _sc_gather_attend(entiEmbs, idx2d, att_p)
    return (uEmbeds, out[:_N_ITEMS])


# speedup vs baseline: 6.5329x; 6.5329x over previous
import functools

import jax
import jax.numpy as jnp
from jax import lax
from jax.experimental import pallas as pl
from jax.experimental.pallas import tpu as pltpu
from jax.experimental.pallas import tpu_sc as plsc

_N_ITEMS = 50000
_EPI = 16
_D = 64
_NW = 32
_C = 32
_NPT = 1568
_NP = _NW * _NPT
_NCH = _NPT // _C


def _tree(vals, op):
    while len(vals) > 1:
        vals = [op(vals[j], vals[j + 1]) for j in range(0, len(vals) - 1, 2)] + (
            [vals[-1]] if len(vals) % 2 else [])
    return vals[0]


def _body(ent_hbm, idx_hbm, att_hbm, out_hbm,
          idx_v, rows_v, att_v, base_v, out_v, sem):
    wid = lax.axis_index("s") * 2 + lax.axis_index("c")

    def chunk(k, carry):
        base = wid * _NPT + k * _C
        pltpu.sync_copy(idx_hbm.at[pl.ds(base * _EPI, _C * _EPI)], idx_v)
        gathers = [
            pltpu.async_copy(ent_hbm.at[idx_v.at[pl.ds(j * 128, 128)]],
                             rows_v.at[pl.ds(j * 128, 128)], sem)
            for j in range(4)
        ]
        pltpu.sync_copy(att_hbm.at[pl.ds(base, _C)], att_v)
        pltpu.sync_copy(ent_hbm.at[pl.ds(base, _C)], base_v)

        for g in gathers:
            g.wait()

        def item(i, c):
            a = att_v[i]
            m = _tree([a[e] for e in range(_EPI)], jnp.maximum)
            ev = jnp.exp(a - m)
            evl = [ev[e] for e in range(_EPI)]
            s = _tree(evl, lambda x, y: x + y)
            w = ev / jnp.broadcast_to(s, (_EPI,))
            wl = [w[e] for e in range(_EPI)]
            acc = [base_v[i, pl.ds(d * 16, 16)] for d in range(4)]
            r0 = i * _EPI
            for e in range(_EPI):
                we = wl[e]
                for d in range(4):
                    acc[d] = acc[d] + we * rows_v[r0 + e, pl.ds(d * 16, 16)]
            for d in range(4):
                out_v[i, pl.ds(d * 16, 16)] = acc[d]
            return c

        lax.fori_loop(0, _C, item, 0)

        pltpu.sync_copy(out_v, out_hbm.at[pl.ds(base, _C)])
        return carry

    lax.fori_loop(0, _NCH, chunk, 0)


@functools.partial(
    pl.kernel,
    out_type=jax.ShapeDtypeStruct((_NP, _D), jnp.float32),
    mesh=plsc.VectorSubcoreMesh(core_axis_name="c", subcore_axis_name="s",
                                num_cores=2, num_subcores=16),
    scratch_types=[
        pltpu.VMEM((_C * _EPI,), jnp.int32),
        pltpu.VMEM((_C * _EPI, _D), jnp.float32),
        pltpu.VMEM((_C, _EPI), jnp.float32),
        pltpu.VMEM((_C, _D), jnp.float32),
        pltpu.VMEM((_C, _D), jnp.float32),
        pltpu.SemaphoreType.DMA,
    ],
    compiler_params=pltpu.CompilerParams(use_tc_tiling_on_sc=False),
)
def _sc_gather_attend(ent, idx, att, out, *scratch):
    _body(ent, idx, att, out, *scratch)


def kernel(item_entities, entiEmbs, att, uEmbeds):
    ie = item_entities.astype(jnp.int32)
    pad = _NP - _N_ITEMS
    ie_p = jnp.pad(ie, ((0, pad), (0, 0)))
    att_p = jnp.pad(att, ((0, pad), (0, 0)))
    idx_flat = ie_p.reshape(_NP * _EPI)
    out = _sc_gather_attend(entiEmbs, idx_flat, att_p)
    return (uEmbeds, out[:_N_ITEMS])


# speedup vs baseline: 9.5970x; 1.4690x over previous
import functools

import jax
import jax.numpy as jnp
from jax import lax
from jax.experimental import pallas as pl
from jax.experimental.pallas import tpu as pltpu
from jax.experimental.pallas import tpu_sc as plsc

_N = 50000
_EPI = 16
_D = 64
_NW = 32
_C = 32
_NFULL = _N // _C
_TAIL = _N - _NFULL * _C
_TB = _NFULL * _C
_EXTRA = _NFULL % _NW


def _tree(vals, op):
    while len(vals) > 1:
        vals = [op(vals[j], vals[j + 1]) for j in range(0, len(vals) - 1, 2)] + (
            [vals[-1]] if len(vals) % 2 else [])
    return vals[0]


def _body(ent, ie, att_h, out_h,
          idx2_v, idx1_v, rows_v, att_v, base_v, out_v,
          sem_g, sem_in, sem_out):
    wid = lax.axis_index("s") * 2 + lax.axis_index("c")
    nch = 48 + jnp.where(wid < _EXTRA, 1, 0)

    def gather_descs(p):
        return [
            pltpu.make_async_copy(
                ent.at[idx1_v.at[pl.ds(p * 512 + j * 128, 128)]],
                rows_v.at[pl.ds(p * 512 + j * 128, 128)], sem_g)
            for j in range(4)
        ]

    def in_descs(base, p):
        return [
            pltpu.make_async_copy(att_h.at[pl.ds(base, _C)],
                                  att_v.at[pl.ds(p * _C, _C)], sem_in),
            pltpu.make_async_copy(ent.at[pl.ds(base, _C)],
                                  base_v.at[pl.ds(p * _C, _C)], sem_in),
        ]

    def out_desc(base, p):
        return pltpu.make_async_copy(out_v.at[pl.ds(p * _C, _C)],
                                     out_h.at[pl.ds(base, _C)], sem_out)

    def stage(c, p):
        base = c * _C
        pltpu.sync_copy(ie.at[pl.ds(base, _C)], idx2_v.at[p])

        def rp(i, cc):
            idx1_v[pl.ds(p * 512 + i * _EPI, _EPI)] = idx2_v[p, i]
            return cc

        lax.fori_loop(0, _C, rp, 0)
        for d in gather_descs(p):
            d.start()
        for d in in_descs(base, p):
            d.start()

    def compute(c, p):
        def item(i, cc):
            a = att_v[p * _C + i]
            m = _tree([a[e] for e in range(_EPI)], jnp.maximum)
            ev = jnp.exp(a - m)
            evl = [ev[e] for e in range(_EPI)]
            s = _tree(evl, lambda x, y: x + y)
            w = ev / jnp.broadcast_to(s, (_EPI,))
            wl = [w[e] for e in range(_EPI)]
            acc = [base_v[p * _C + i, pl.ds(d * 16, 16)] for d in range(4)]
            r0 = p * 512 + i * _EPI
            for e in range(_EPI):
                we = wl[e]
                for d in range(4):
                    acc[d] = acc[d] + we * rows_v[r0 + e, pl.ds(d * 16, 16)]
            for d in range(4):
                out_v[p * _C + i, pl.ds(d * 16, 16)] = acc[d]
            return cc

        lax.fori_loop(0, _C, item, 0)

    stage(wid, 0)

    def step(k, carry):
        p = lax.rem(k, 2)
        p1 = 1 - p

        @pl.when(k + 1 < nch)
        def _():
            stage(wid + _NW * (k + 1), p1)

        c = wid + _NW * k
        base = c * _C
        for d in gather_descs(p):
            d.wait()
        for d in in_descs(base, p):
            d.wait()
        compute(c, p)

        @pl.when(k >= 1)
        def _():
            out_desc(base, p).wait()

        out_desc(base, p).start()
        return carry

    lax.fori_loop(0, nch, step, 0)

    out_desc(0, 0).wait()

    @pl.when(wid == _NW - 1)
    def _():
        pltpu.sync_copy(ie.at[pl.ds(_TB, _TAIL)], idx2_v.at[0, pl.ds(0, _TAIL)])

        def rp(i, cc):
            idx1_v[pl.ds(i * _EPI, _EPI)] = idx2_v[0, i]
            return cc

        lax.fori_loop(0, _TAIL, rp, 0)
        for j in range(2):
            pltpu.async_copy(
                ent.at[idx1_v.at[pl.ds(j * 128, 128)]],
                rows_v.at[pl.ds(j * 128, 128)], sem_g).wait()
        pltpu.sync_copy(att_h.at[pl.ds(_TB, _TAIL)], att_v.at[pl.ds(0, _TAIL)])
        pltpu.sync_copy(ent.at[pl.ds(_TB, _TAIL)], base_v.at[pl.ds(0, _TAIL)])

        def item(i, cc):
            a = att_v[i]
            m = _tree([a[e] for e in range(_EPI)], jnp.maximum)
            ev = jnp.exp(a - m)
            evl = [ev[e] for e in range(_EPI)]
            s = _tree(evl, lambda x, y: x + y)
            w = ev / jnp.broadcast_to(s, (_EPI,))
            wl = [w[e] for e in range(_EPI)]
            acc = [base_v[i, pl.ds(d * 16, 16)] for d in range(4)]
            r0 = i * _EPI
            for e in range(_EPI):
                we = wl[e]
                for d in range(4):
                    acc[d] = acc[d] + we * rows_v[r0 + e, pl.ds(d * 16, 16)]
            for d in range(4):
                out_v[i, pl.ds(d * 16, 16)] = acc[d]
            return cc

        lax.fori_loop(0, _TAIL, item, 0)
        pltpu.sync_copy(out_v.at[pl.ds(0, _TAIL)], out_h.at[pl.ds(_TB, _TAIL)])


@functools.partial(
    pl.kernel,
    out_type=jax.ShapeDtypeStruct((_N, _D), jnp.float32),
    mesh=plsc.VectorSubcoreMesh(core_axis_name="c", subcore_axis_name="s",
                                num_cores=2, num_subcores=16),
    scratch_types=[
        pltpu.VMEM((2, _C, _EPI), jnp.int32),
        pltpu.VMEM((2 * 512,), jnp.int32),
        pltpu.VMEM((2 * 512, _D), jnp.float32),
        pltpu.VMEM((2 * _C, _EPI), jnp.float32),
        pltpu.VMEM((2 * _C, _D), jnp.float32),
        pltpu.VMEM((2 * _C, _D), jnp.float32),
        pltpu.SemaphoreType.DMA,
        pltpu.SemaphoreType.DMA,
        pltpu.SemaphoreType.DMA,
    ],
    compiler_params=pltpu.CompilerParams(use_tc_tiling_on_sc=False),
)
def _sc_gather_attend(ent, ie, att, out, *scratch):
    _body(ent, ie, att, out, *scratch)


def kernel(item_entities, entiEmbs, att, uEmbeds):
    ie = item_entities.astype(jnp.int32)
    out = _sc_gather_attend(entiEmbs, ie, att)
    return (uEmbeds, out)


# speedup vs baseline: 9.7804x; 1.0191x over previous
import functools

import jax
import jax.numpy as jnp
from jax import lax
from jax.experimental import pallas as pl
from jax.experimental.pallas import tpu as pltpu
from jax.experimental.pallas import tpu_sc as plsc

_N = 50000
_EPI = 16
_D = 64
_NW = 32
_C = 32
_CI = _C * _EPI
_NFULL = _N // _C
_TAIL = _N - _NFULL * _C
_TB = _NFULL * _C
_EXTRA = _NFULL % _NW


def _tree(vals, op):
    while len(vals) > 1:
        vals = [op(vals[j], vals[j + 1]) for j in range(0, len(vals) - 1, 2)] + (
            [vals[-1]] if len(vals) % 2 else [])
    return vals[0]


def _body(ent, ie, att_h, out_h,
          idx_v, rows_v, att_v, base_v, out_v,
          sem_g, sem_in, sem_out):
    wid = lax.axis_index("s") * 2 + lax.axis_index("c")
    nch = 48 + jnp.where(wid < _EXTRA, 1, 0)

    def gather_descs(p):
        return [
            pltpu.make_async_copy(
                ent.at[idx_v.at[pl.ds(p * _CI + j * 128, 128)]],
                rows_v.at[pl.ds(p * _CI + j * 128, 128)], sem_g)
            for j in range(4)
        ]

    def in_descs(base, p):
        return [
            pltpu.make_async_copy(att_h.at[pl.ds(base * _EPI, _CI)],
                                  att_v.at[pl.ds(p * _CI, _CI)], sem_in),
            pltpu.make_async_copy(ent.at[pl.ds(base, _C)],
                                  base_v.at[pl.ds(p * _C, _C)], sem_in),
        ]

    def out_desc(base, p):
        return pltpu.make_async_copy(out_v.at[pl.ds(p * _C * _D, _C * _D)],
                                     out_h.at[pl.ds(base * _D, _C * _D)], sem_out)

    def stage(c, p):
        base = c * _C
        pltpu.sync_copy(ie.at[pl.ds(base * _EPI, _CI)],
                        idx_v.at[pl.ds(p * _CI, _CI)])
        for d in gather_descs(p):
            d.start()
        for d in in_descs(base, p):
            d.start()

    def compute(p):
        def item(i, cc):
            a = att_v[pl.ds(p * _CI + i * _EPI, _EPI)]
            m = _tree([a[e] for e in range(_EPI)], jnp.maximum)
            ev = jnp.exp(a - m)
            evl = [ev[e] for e in range(_EPI)]
            s = _tree(evl, lambda x, y: x + y)
            w = ev / jnp.broadcast_to(s, (_EPI,))
            wl = [w[e] for e in range(_EPI)]
            acc = [base_v[p * _C + i, pl.ds(d * 16, 16)] for d in range(4)]
            r0 = p * _CI + i * _EPI
            for e in range(_EPI):
                we = wl[e]
                for d in range(4):
                    acc[d] = acc[d] + we * rows_v[r0 + e, pl.ds(d * 16, 16)]
            o0 = (p * _C + i) * _D
            for d in range(4):
                out_v[pl.ds(o0 + d * 16, 16)] = acc[d]
            return cc

        lax.fori_loop(0, _C, item, 0)

    stage(wid, 0)

    def step(k, carry):
        p = lax.rem(k, 2)
        p1 = 1 - p

        @pl.when(k + 1 < nch)
        def _():
            stage(wid + _NW * (k + 1), p1)

        base = (wid + _NW * k) * _C
        for d in gather_descs(p):
            d.wait()
        for d in in_descs(base, p):
            d.wait()
        compute(p)

        @pl.when(k >= 1)
        def _():
            out_desc(base, p).wait()

        out_desc(base, p).start()
        return carry

    lax.fori_loop(0, nch, step, 0)

    out_desc(0, 0).wait()

    @pl.when(wid == _NW - 1)
    def _():
        nti = _TAIL * _EPI
        pltpu.sync_copy(ie.at[pl.ds(_TB * _EPI, nti)], idx_v.at[pl.ds(0, nti)])
        for j in range(2):
            pltpu.async_copy(
                ent.at[idx_v.at[pl.ds(j * 128, 128)]],
                rows_v.at[pl.ds(j * 128, 128)], sem_g).wait()
        pltpu.sync_copy(att_h.at[pl.ds(_TB * _EPI, nti)], att_v.at[pl.ds(0, nti)])
        pltpu.sync_copy(ent.at[pl.ds(_TB, _TAIL)], base_v.at[pl.ds(0, _TAIL)])

        def item(i, cc):
            a = att_v[pl.ds(i * _EPI, _EPI)]
            m = _tree([a[e] for e in range(_EPI)], jnp.maximum)
            ev = jnp.exp(a - m)
            evl = [ev[e] for e in range(_EPI)]
            s = _tree(evl, lambda x, y: x + y)
            w = ev / jnp.broadcast_to(s, (_EPI,))
            wl = [w[e] for e in range(_EPI)]
            acc = [base_v[i, pl.ds(d * 16, 16)] for d in range(4)]
            r0 = i * _EPI
            for e in range(_EPI):
                we = wl[e]
                for d in range(4):
                    acc[d] = acc[d] + we * rows_v[r0 + e, pl.ds(d * 16, 16)]
            for d in range(4):
                out_v[pl.ds(i * _D + d * 16, 16)] = acc[d]
            return cc

        lax.fori_loop(0, _TAIL, item, 0)
        pltpu.sync_copy(out_v.at[pl.ds(0, _TAIL * _D)],
                        out_h.at[pl.ds(_TB * _D, _TAIL * _D)])


@functools.partial(
    pl.kernel,
    out_type=jax.ShapeDtypeStruct((_N * _D,), jnp.float32),
    mesh=plsc.VectorSubcoreMesh(core_axis_name="c", subcore_axis_name="s",
                                num_cores=2, num_subcores=16),
    scratch_types=[
        pltpu.VMEM((2 * _CI,), jnp.int32),
        pltpu.VMEM((2 * _CI, _D), jnp.float32),
        pltpu.VMEM((2 * _CI,), jnp.float32),
        pltpu.VMEM((2 * _C, _D), jnp.float32),
        pltpu.VMEM((2 * _C * _D,), jnp.float32),
        pltpu.SemaphoreType.DMA,
        pltpu.SemaphoreType.DMA,
        pltpu.SemaphoreType.DMA,
    ],
    compiler_params=pltpu.CompilerParams(use_tc_tiling_on_sc=False),
)
def _sc_gather_attend(ent, ie, att, out, *scratch):
    _body(ent, ie, att, out, *scratch)


def kernel(item_entities, entiEmbs, att, uEmbeds):
    ie_flat = item_entities.astype(jnp.int32).reshape(_N * _EPI)
    att_flat = att.reshape(_N * _EPI)
    out = _sc_gather_attend(entiEmbs, ie_flat, att_flat)
    return (uEmbeds, out.reshape(_N, _D))


# speedup vs baseline: 11.9245x; 1.2192x over previous
import functools

import jax
import jax.numpy as jnp
from jax import lax
from jax.experimental import pallas as pl
from jax.experimental.pallas import tpu as pltpu
from jax.experimental.pallas import tpu_sc as plsc

_N = 50000
_NE = 200001
_EPI = 16
_D = 64
_DW = _D // 2
_NW = 32
_C = 64
_CI = _C * _EPI
_NFULL = _N // _C
_TAIL = _N - _NFULL * _C
_TB = _NFULL * _C
_EXTRA = _NFULL % _NW
_KMIN = _NFULL // _NW


def _tree(vals, op):
    while len(vals) > 1:
        vals = [op(vals[j], vals[j + 1]) for j in range(0, len(vals) - 1, 2)] + (
            [vals[-1]] if len(vals) % 2 else [])
    return vals[0]


_HIMASK = jnp.int32(-65536)


def _unpack(xi):
    lo = lax.bitcast_convert_type(lax.shift_left(xi, 16), jnp.float32)
    hi = lax.bitcast_convert_type(lax.bitwise_and(xi, _HIMASK), jnp.float32)
    return lo, hi


def _body(tab, ie_t, att_t, out_h,
          idx_v, rows_v, att_v, w_v, base_v, out_v,
          sem_g, sem_in, sem_idx, sem_out):
    wid = lax.axis_index("s") * 2 + lax.axis_index("c")
    nch = _KMIN + jnp.where(wid < _EXTRA, 1, 0)

    def idx_desc(base, p):
        return pltpu.make_async_copy(ie_t.at[:, pl.ds(base, _C)],
                                     idx_v.at[p], sem_idx)

    def gather_descs(p):
        return [
            pltpu.make_async_copy(
                tab.at[idx_v.at[p, e]],
                rows_v.at[pl.ds(p * _CI + e * _C, _C)], sem_g)
            for e in range(_EPI)
        ]

    def in_descs(base, p):
        return [
            pltpu.make_async_copy(att_t.at[:, pl.ds(base, _C)],
                                  att_v.at[p], sem_in),
            pltpu.make_async_copy(tab.at[pl.ds(base, _C)],
                                  base_v.at[pl.ds(p * _C, _C)], sem_in),
        ]

    def out_desc(base, p):
        return pltpu.make_async_copy(out_v.at[pl.ds(p * _C * _D, _C * _D)],
                                     out_h.at[pl.ds(base * _D, _C * _D)], sem_out)

    def softmax(p, nit):
        for h in range(nit // 16):
            aes = [att_v[p, e, pl.ds(h * 16, 16)] for e in range(_EPI)]
            m = _tree(aes, jnp.maximum)
            evs = [jnp.exp(a - m) for a in aes]
            s = _tree(evs, lambda x, y: x + y)
            inv = 1.0 / s
            for e in range(_EPI):
                w_v[pl.ds(p * _CI + e * nit + h * 16, 16)] = evs[e] * inv

    def item_block(p, nit):
        def item(i, cc):
            b0 = p * _C + i
            acc = []
            for g in range(2):
                xi = base_v[b0, pl.ds(g * 16, 16)]
                acc += _unpack(xi)
            r0 = p * _CI + i
            w0 = p * _CI + i
            for e in range(_EPI):
                we = w_v[pl.ds(w0 + e * nit, 16)][0]
                r = r0 + e * nit
                for g in range(2):
                    lo, hi = _unpack(rows_v[r, pl.ds(g * 16, 16)])
                    acc[2 * g] = acc[2 * g] + we * lo
                    acc[2 * g + 1] = acc[2 * g + 1] + we * hi
            o0 = (p * _C + i) * _D
            for q in range(4):
                out_v[pl.ds(o0 + q * 16, 16)] = acc[q]
            return cc
        return item

    def compute(p):
        softmax(p, _C)
        lax.fori_loop(0, _C, item_block(p, _C), 0)

    idx_desc(wid * _C, 0).start()
    idx_desc(wid * _C, 0).wait()
    for d in gather_descs(0):
        d.start()
    for d in in_descs(wid * _C, 0):
        d.start()

    @pl.when(nch > 1)
    def _():
        idx_desc((wid + _NW) * _C, 1).start()

    def step(k, carry):
        p = lax.rem(k, 2)
        p1 = 1 - p
        base = (wid + _NW * k) * _C

        for d in gather_descs(p):
            d.wait()
        for d in in_descs(base, p):
            d.wait()

        @pl.when(k + 1 < nch)
        def _():
            base1 = (wid + _NW * (k + 1)) * _C
            idx_desc(base1, p1).wait()
            for d in gather_descs(p1):
                d.start()
            for d in in_descs(base1, p1):
                d.start()

        @pl.when(k + 2 < nch)
        def _():
            idx_desc((wid + _NW * (k + 2)) * _C, p).start()

        compute(p)

        @pl.when(k >= 1)
        def _():
            out_desc(base, p).wait()

        out_desc(base, p).start()
        return carry

    lax.fori_loop(0, nch, step, 0)

    out_desc(0, 0).wait()

    @pl.when(wid == _NW - 1)
    def _():
        pltpu.sync_copy(ie_t.at[:, pl.ds(_TB, _TAIL)],
                        idx_v.at[0, :, pl.ds(0, _TAIL)])
        for e in range(_EPI):
            pltpu.async_copy(
                tab.at[idx_v.at[0, e, pl.ds(0, _TAIL)]],
                rows_v.at[pl.ds(e * _TAIL, _TAIL)], sem_g).wait()
        pltpu.sync_copy(att_t.at[:, pl.ds(_TB, _TAIL)],
                        att_v.at[0, :, pl.ds(0, _TAIL)])
        pltpu.sync_copy(tab.at[pl.ds(_TB, _TAIL)], base_v.at[pl.ds(0, _TAIL)])
        softmax(0, _TAIL)
        lax.fori_loop(0, _TAIL, item_block(0, _TAIL), 0)
        pltpu.sync_copy(out_v.at[pl.ds(0, _TAIL * _D)],
                        out_h.at[pl.ds(_TB * _D, _TAIL * _D)])


@functools.partial(
    pl.kernel,
    out_type=jax.ShapeDtypeStruct((_N * _D,), jnp.float32),
    mesh=plsc.VectorSubcoreMesh(core_axis_name="c", subcore_axis_name="s",
                                num_cores=2, num_subcores=16),
    scratch_types=[
        pltpu.VMEM((2, _EPI, _C), jnp.int32),
        pltpu.VMEM((2 * _CI, _DW), jnp.int32),
        pltpu.VMEM((2, _EPI, _C), jnp.float32),
        pltpu.VMEM((2 * _CI + 16,), jnp.float32),
        pltpu.VMEM((2 * _C, _DW), jnp.int32),
        pltpu.VMEM((2 * _C * _D,), jnp.float32),
        pltpu.SemaphoreType.DMA,
        pltpu.SemaphoreType.DMA,
        pltpu.SemaphoreType.DMA,
        pltpu.SemaphoreType.DMA,
    ],
    compiler_params=pltpu.CompilerParams(use_tc_tiling_on_sc=False),
)
def _sc_gather_attend(tab, ie_t, att_t, out, *scratch):
    _body(tab, ie_t, att_t, out, *scratch)


def kernel(item_entities, entiEmbs, att, uEmbeds):
    tab_perm = (entiEmbs.reshape(_NE, 2, 2, 16)
                .transpose(0, 1, 3, 2)
                .reshape(_NE, _D)
                .astype(jnp.bfloat16))
    tab_i32 = lax.bitcast_convert_type(tab_perm.reshape(_NE, _DW, 2),
                                       jnp.int32)
    ie_t = item_entities.astype(jnp.int32).T
    att_t = att.T
    out = _sc_gather_attend(tab_i32, ie_t, att_t)
    return (uEmbeds, out.reshape(_N, _D))


# speedup vs baseline: 13.4656x; 1.1292x over previous
import functools

import jax
import jax.numpy as jnp
from jax import lax
from jax.experimental import pallas as pl
from jax.experimental.pallas import tpu as pltpu
from jax.experimental.pallas import tpu_sc as plsc

_N = 50000
_NE = 200001
_EPI = 16
_D = 64
_DW = _D // 2
_NW = 32
_C = 80
_CI = _C * _EPI
_NFULL = _N // _C
_TAIL = _N - _NFULL * _C
_TB = _NFULL * _C
_EXTRA = _NFULL % _NW
_KMIN = _NFULL // _NW


def _tree(vals, op):
    while len(vals) > 1:
        vals = [op(vals[j], vals[j + 1]) for j in range(0, len(vals) - 1, 2)] + (
            [vals[-1]] if len(vals) % 2 else [])
    return vals[0]


def _unpack(xi):
    lo = lax.bitcast_convert_type(lax.shift_left(xi, 16), jnp.float32)
    hi = lax.bitcast_convert_type(xi, jnp.float32)
    return lo, hi


def _body(tab, ie_t, att_t, out_h,
          idx_v, rows_v, att_v, w_v, base_v, out_v,
          sem_g, sem_in, sem_idx, sem_out):
    wid = lax.axis_index("s") * 2 + lax.axis_index("c")
    nch = _KMIN + jnp.where(wid < _EXTRA, 1, 0)

    def idx_desc(base, p):
        return pltpu.make_async_copy(ie_t.at[:, pl.ds(base, _C)],
                                     idx_v.at[p], sem_idx)

    def gather_descs(p):
        return [
            pltpu.make_async_copy(
                tab.at[idx_v.at[p, e]],
                rows_v.at[pl.ds(p * _CI + e * _C, _C)], sem_g)
            for e in range(_EPI)
        ]

    def in_descs(base, p):
        return [
            pltpu.make_async_copy(att_t.at[:, pl.ds(base, _C)],
                                  att_v.at[p], sem_in),
            pltpu.make_async_copy(tab.at[pl.ds(base, _C)],
                                  base_v.at[pl.ds(p * _C, _C)], sem_in),
        ]

    def out_desc(base, p):
        return pltpu.make_async_copy(out_v.at[pl.ds(p * _C, _C)],
                                     out_h.at[pl.ds(base, _C)], sem_out)

    def softmax(p, nit):
        for h in range(nit // 16):
            aes = [att_v[p, e, pl.ds(h * 16, 16)] for e in range(_EPI)]
            m = _tree(aes, jnp.maximum)
            evs = [jnp.exp(a - m) for a in aes]
            s = _tree(evs, lambda x, y: x + y)
            inv = 1.0 / s
            for e in range(_EPI):
                w_v[pl.ds(p * _CI + e * nit + h * 16, 16)] = evs[e] * inv

    def item_block(p, nit):
        def item(i, cc):
            b0 = p * _C + i
            acc = []
            for g in range(2):
                xi = base_v[b0, pl.ds(g * 16, 16)]
                acc += _unpack(xi)
            r0 = p * _CI + i
            w0 = p * _CI + i
            for e in range(_EPI):
                we = w_v[pl.ds(w0 + e * nit, 16)][0]
                r = r0 + e * nit
                for g in range(2):
                    lo, hi = _unpack(rows_v[r, pl.ds(g * 16, 16)])
                    acc[2 * g] = acc[2 * g] + we * lo
                    acc[2 * g + 1] = acc[2 * g + 1] + we * hi
            for q in range(4):
                out_v[p * _C + i, pl.ds(q * 16, 16)] = acc[q]
            return cc
        return item

    def compute(p):
        softmax(p, _C)
        lax.fori_loop(0, _C, item_block(p, _C), 0)

    idx_desc(wid * _C, 0).start()
    idx_desc(wid * _C, 0).wait()
    for d in gather_descs(0):
        d.start()
    for d in in_descs(wid * _C, 0):
        d.start()

    @pl.when(nch > 1)
    def _():
        idx_desc((wid + _NW) * _C, 1).start()

    def step(k, carry):
        p = lax.rem(k, 2)
        p1 = 1 - p
        base = (wid + _NW * k) * _C

        for d in gather_descs(p):
            d.wait()
        for d in in_descs(base, p):
            d.wait()

        @pl.when(k + 1 < nch)
        def _():
            base1 = (wid + _NW * (k + 1)) * _C
            idx_desc(base1, p1).wait()
            for d in gather_descs(p1):
                d.start()
            for d in in_descs(base1, p1):
                d.start()

        @pl.when(k + 2 < nch)
        def _():
            idx_desc((wid + _NW * (k + 2)) * _C, p).start()

        compute(p)

        @pl.when(k >= 1)
        def _():
            out_desc(base, p).wait()

        out_desc(base, p).start()
        return carry

    lax.fori_loop(0, nch, step, 0)

    out_desc(0, 0).wait()

    if not _TAIL:
        return

    @pl.when(wid == _NW - 1)
    def _():
        pltpu.sync_copy(ie_t.at[:, pl.ds(_TB, _TAIL)],
                        idx_v.at[0, :, pl.ds(0, _TAIL)])
        for e in range(_EPI):
            pltpu.async_copy(
                tab.at[idx_v.at[0, e, pl.ds(0, _TAIL)]],
                rows_v.at[pl.ds(e * _TAIL, _TAIL)], sem_g).wait()
        pltpu.sync_copy(att_t.at[:, pl.ds(_TB, _TAIL)],
                        att_v.at[0, :, pl.ds(0, _TAIL)])
        pltpu.sync_copy(tab.at[pl.ds(_TB, _TAIL)], base_v.at[pl.ds(0, _TAIL)])
        softmax(0, _TAIL)
        lax.fori_loop(0, _TAIL, item_block(0, _TAIL), 0)
        pltpu.sync_copy(out_v.at[pl.ds(0, _TAIL)],
                        out_h.at[pl.ds(_TB, _TAIL)])


@functools.partial(
    pl.kernel,
    out_type=jax.ShapeDtypeStruct((_N, _D), jnp.float32),
    mesh=plsc.VectorSubcoreMesh(core_axis_name="c", subcore_axis_name="s",
                                num_cores=2, num_subcores=16),
    scratch_types=[
        pltpu.VMEM((2, _EPI, _C), jnp.int32),
        pltpu.VMEM((2 * _CI, _DW), jnp.int32),
        pltpu.VMEM((2, _EPI, _C), jnp.float32),
        pltpu.VMEM((2 * _CI + 16,), jnp.float32),
        pltpu.VMEM((2 * _C, _DW), jnp.int32),
        pltpu.VMEM((2 * _C, _D), jnp.float32),
        pltpu.SemaphoreType.DMA,
        pltpu.SemaphoreType.DMA,
        pltpu.SemaphoreType.DMA,
        pltpu.SemaphoreType.DMA,
    ],
    compiler_params=pltpu.CompilerParams(use_tc_tiling_on_sc=False),
)
def _sc_gather_attend(tab, ie_t, att_t, out, *scratch):
    _body(tab, ie_t, att_t, out, *scratch)


def kernel(item_entities, entiEmbs, att, uEmbeds):
    xi = lax.bitcast_convert_type(entiEmbs, jnp.uint32).reshape(_NE, 2, 2, 16)
    rb = (xi + jnp.uint32(0x7FFF) + ((xi >> jnp.uint32(16)) & jnp.uint32(1))
          ) >> jnp.uint32(16)
    packed = rb[:, :, 0, :] | (rb[:, :, 1, :] << jnp.uint32(16))
    tab_i32 = lax.bitcast_convert_type(packed.reshape(_NE, _DW), jnp.int32)
    ie_t = item_entities.astype(jnp.int32).T
    att_t = att.T
    out = _sc_gather_attend(tab_i32, ie_t, att_t)
    return (uEmbeds, out)
